# baseline (speedup 1.0000x reference)
import functools

import jax
import jax.numpy as jnp
import numpy as np
from jax import lax
from jax.experimental import pallas as pl
from jax.experimental.pallas import tpu as pltpu
from jax.experimental.pallas import tpu_sc as plsc

EMB = 128
TOTAL_ROWS = 4096 * 200
QUAD = 4
NQ = TOTAL_ROWS // QUAD
QROW = QUAD * EMB
NPAT = 81
NPAD = 88
CHQ = 64
NB = 2


def _make_sc_lookup():
    info = plsc.get_sparse_core_info()
    nc, ns = info.num_cores, info.num_subcores
    nw = nc * ns
    quads_per_w = NQ // nw
    n_steps = quads_per_w // CHQ
    n_groups = n_steps // NB

    mesh = plsc.VectorSubcoreMesh(core_axis_name="c", subcore_axis_name="s")

    @functools.partial(
        pl.kernel,
        mesh=mesh,
        out_type=jax.ShapeDtypeStruct((NQ, QUAD, EMB), jnp.float32),
        scratch_types=[
            pltpu.VMEM_SHARED((ns * NPAD, QUAD, EMB), jnp.float32),
            pltpu.VMEM((n_steps, CHQ), jnp.int32),
            pltpu.VMEM((NB, CHQ, QUAD, EMB), jnp.float32),
            pltpu.SemaphoreType.DMA,
        ] + [pltpu.SemaphoreType.DMA] * NB,
    )
    def k(idx_hbm, qtab_hbm, out_hbm, qtab_sh, idx_v, rows_v, gsem, *wsems):
        c = lax.axis_index("c")
        s = lax.axis_index("s")
        wid = s * nc + c
        base = wid * quads_per_w
        pltpu.sync_copy(qtab_hbm, qtab_sh.at[pl.ds(s * NPAD, NPAD)])
        pltpu.sync_copy(idx_hbm.at[wid], idx_v)

        def group(g, carry):
            for b in range(NB):
                st = g * NB + b

                @pl.when(g > 0)
                def _():
                    pltpu.make_async_copy(
                        rows_v.at[b], out_hbm.at[pl.ds(base, CHQ)], wsems[b]
                    ).wait()

                pltpu.async_copy(
                    qtab_sh.at[idx_v.at[st]], rows_v.at[b], gsem
                ).wait()
                pltpu.async_copy(
                    rows_v.at[b], out_hbm.at[pl.ds(base + st * CHQ, CHQ)], wsems[b]
                )
            return carry

        lax.fori_loop(0, n_groups, group, 0)
        for b in range(NB):
            pltpu.make_async_copy(
                rows_v.at[b], out_hbm.at[pl.ds(base, CHQ)], wsems[b]
            ).wait()

    return k, nc, nw, quads_per_w, n_steps


_sc_lookup, _NC, _NW, _QPW, _NSTEPS = _make_sc_lookup()

_PAT = np.stack(
    [np.arange(NPAT) // 27, (np.arange(NPAT) // 9) % 3,
     (np.arange(NPAT) // 3) % 3, np.arange(NPAT) % 3], axis=1
).reshape(-1)


def kernel(indices, table):
    B, T = indices.shape
    q = indices.reshape(NQ, QUAD).astype(jnp.int32)
    qidx = q[:, 0] * 27 + q[:, 1] * 9 + q[:, 2] * 3 + q[:, 3]
    sub = (jnp.arange(NQ, dtype=jnp.int32) // _QPW) // _NC
    qidx = qidx + sub * NPAD
    qtab = jnp.take(table, _PAT, axis=0).reshape(NPAT, QUAD, EMB)
    qtab = jnp.concatenate([qtab, jnp.zeros((NPAD - NPAT, QUAD, EMB), jnp.float32)], axis=0)
    out = _sc_lookup(qidx.reshape(_NW, _NSTEPS, CHQ), qtab)
    return out.reshape(B, T, EMB)


# speedup vs baseline: 1.7477x; 1.7477x over previous
import functools

import jax
import jax.numpy as jnp
from jax import lax
from jax.experimental import pallas as pl
from jax.experimental.pallas import tpu as pltpu
from jax.experimental.pallas import tpu_sc as plsc

EMB = 128
TOTAL_ROWS = 4096 * 200
CH = 128
NB = 4
RPAD = 8


def _make_sc_lookup(total_rows, emb):
    info = plsc.get_sparse_core_info()
    nc, ns = info.num_cores, info.num_subcores
    nw = nc * ns
    rows_per_w = total_rows // nw
    n_steps = rows_per_w // CH
    n_groups = n_steps // NB

    mesh = plsc.VectorSubcoreMesh(core_axis_name="c", subcore_axis_name="s")

    @functools.partial(
        pl.kernel,
        mesh=mesh,
        out_type=jax.ShapeDtypeStruct((total_rows, emb), jnp.float32),
        scratch_types=[
            pltpu.VMEM_SHARED((ns * RPAD, emb), jnp.float32),
            pltpu.VMEM((n_steps, CH), jnp.int32),
            pltpu.VMEM((NB, CH, emb), jnp.float32),
            pltpu.SemaphoreType.DMA,
        ] + [pltpu.SemaphoreType.DMA] * NB,
    )
    def k(idx_hbm, table_hbm, out_hbm, table_sh, idx_v, rows_v, gsem, *wsems):
        c = lax.axis_index("c")
        s = lax.axis_index("s")
        wid = s * nc + c
        base = wid * rows_per_w
        pltpu.sync_copy(table_hbm, table_sh.at[pl.ds(s * RPAD, RPAD)])
        pltpu.sync_copy(idx_hbm.at[wid], idx_v)

        def group(g, carry):
            for b in range(NB):
                st = g * NB + b

                @pl.when(g > 0)
                def _():
                    pltpu.make_async_copy(
                        rows_v.at[b], out_hbm.at[pl.ds(base, CH)], wsems[b]
                    ).wait()

                pltpu.async_copy(table_sh.at[idx_v.at[st]], rows_v.at[b], gsem).wait()
                pltpu.async_copy(
                    rows_v.at[b], out_hbm.at[pl.ds(base + st * CH, CH)], wsems[b]
                )
            return carry

        lax.fori_loop(0, n_groups, group, 0)
        for b in range(NB):
            pltpu.make_async_copy(
                rows_v.at[b], out_hbm.at[pl.ds(base, CH)], wsems[b]
            ).wait()

    return k, nc, nw, rows_per_w, n_steps


_sc_lookup, _NC, _NW, _RPW, _NSTEPS = _make_sc_lookup(TOTAL_ROWS, EMB)


def kernel(indices, table):
    B, T = indices.shape
    flat = indices.reshape(B * T).astype(jnp.int32)
    sub = (jnp.arange(B * T, dtype=jnp.int32) // _RPW) // _NC
    flat = flat + sub * RPAD
    table_p = jnp.concatenate(
        [table, jnp.zeros((RPAD - 3, EMB), jnp.float32)], axis=0
    )
    out = _sc_lookup(flat.reshape(_NW, _NSTEPS, CH), table_p)
    return out.reshape(B, T, EMB)


# speedup vs baseline: 1.8373x; 1.0513x over previous
import functools

import jax
import jax.numpy as jnp
from jax import lax
from jax.experimental import pallas as pl
from jax.experimental.pallas import tpu as pltpu
from jax.experimental.pallas import tpu_sc as plsc

EMB = 128
TOTAL_ROWS = 4096 * 200
CH = 128
NB = 4
GA = 2
RPAD = 8


def _make_sc_lookup(total_rows, emb):
    info = plsc.get_sparse_core_info()
    nc, ns = info.num_cores, info.num_subcores
    nw = nc * ns
    rows_per_w = total_rows // nw
    n_steps = rows_per_w // CH
    n_groups = n_steps // NB

    mesh = plsc.VectorSubcoreMesh(core_axis_name="c", subcore_axis_name="s")

    @functools.partial(
        pl.kernel,
        mesh=mesh,
        out_type=jax.ShapeDtypeStruct((total_rows, emb), jnp.float32),
        scratch_types=[
            pltpu.VMEM_SHARED((ns * RPAD, emb), jnp.float32),
            pltpu.VMEM((n_steps, CH), jnp.int32),
            pltpu.VMEM((NB, CH, emb), jnp.float32),
            pltpu.SemaphoreType.DMA,
        ] + [pltpu.SemaphoreType.DMA] * NB,
    )
    def k(idx_hbm, table_hbm, out_hbm, table_sh, idx_v, rows_v, gsem, *wsems):
        c = lax.axis_index("c")
        s = lax.axis_index("s")
        wid = s * nc + c
        base = wid * rows_per_w
        pltpu.sync_copy(table_hbm, table_sh.at[pl.ds(s * RPAD, RPAD)])
        pltpu.sync_copy(idx_hbm.at[wid], idx_v)

        def gather(step, buf):
            pltpu.async_copy(table_sh.at[idx_v.at[step]], rows_v.at[buf], gsem)

        def wait_gather(buf):
            pltpu.make_async_copy(
                table_sh.at[idx_v.at[0]], rows_v.at[buf], gsem
            ).wait()

        def wait_write(buf):
            pltpu.make_async_copy(
                rows_v.at[buf], out_hbm.at[pl.ds(base, CH)], wsems[buf]
            ).wait()

        for p in range(GA):
            gather(p, p)

        def group(g, carry):
            for b in range(NB):
                st = g * NB + b

                nb = (b + GA) % NB

                @pl.when(st + GA < n_steps)
                def _():

                    @pl.when(st + GA >= NB)
                    def _():
                        wait_write(nb)

                    gather(st + GA, nb)

                wait_gather(b)
                pltpu.async_copy(
                    rows_v.at[b], out_hbm.at[pl.ds(base + st * CH, CH)], wsems[b]
                )
            return carry

        lax.fori_loop(0, n_groups, group, 0)
        for b in range(NB):
            wait_write(b)

    return k, nc, nw, rows_per_w, n_steps


_sc_lookup, _NC, _NW, _RPW, _NSTEPS = _make_sc_lookup(TOTAL_ROWS, EMB)


def kernel(indices, table):
    B, T = indices.shape
    flat = indices.reshape(B * T).astype(jnp.int32)
    sub = (jnp.arange(B * T, dtype=jnp.int32) // _RPW) // _NC
    flat = flat + sub * RPAD
    table_p = jnp.concatenate(
        [table, jnp.zeros((RPAD - 3, EMB), jnp.float32)], axis=0
    )
    out = _sc_lookup(flat.reshape(_NW, _NSTEPS, CH), table_p)
    return out.reshape(B, T, EMB)


# speedup vs baseline: 1.9297x; 1.0503x over previous
import functools

import jax
import jax.numpy as jnp
from jax import lax
from jax.experimental import pallas as pl
from jax.experimental.pallas import tpu as pltpu
from jax.experimental.pallas import tpu_sc as plsc

EMB = 128
TOTAL_ROWS = 4096 * 200
CH = 128
NB = 4
GA = 2


def _make_sc_lookup(total_rows, emb):
    info = plsc.get_sparse_core_info()
    nc, ns = info.num_cores, info.num_subcores
    nw = nc * ns
    rows_per_w = total_rows // nw
    n_steps = rows_per_w // CH
    n_groups = n_steps // NB

    mesh = plsc.VectorSubcoreMesh(core_axis_name="c", subcore_axis_name="s")

    @functools.partial(
        pl.kernel,
        mesh=mesh,
        out_type=jax.ShapeDtypeStruct((total_rows, emb), jnp.float32),
        scratch_types=[
            pltpu.VMEM_SHARED((8, emb), jnp.float32),
            pltpu.VMEM((n_steps, CH), jnp.int32),
            pltpu.VMEM((NB, CH, emb), jnp.float32),
            pltpu.SemaphoreType.DMA,
        ] + [pltpu.SemaphoreType.DMA] * NB,
    )
    def k(idx_hbm, table_hbm, out_hbm, table_sh, idx_v, rows_v, gsem, *wsems):
        c = lax.axis_index("c")
        s = lax.axis_index("s")
        wid = s * nc + c
        base = wid * rows_per_w
        @pl.when(s == 0)
        def _():
            pltpu.sync_copy(table_hbm, table_sh.at[pl.ds(0, 3)])

        pltpu.sync_copy(idx_hbm.at[wid], idx_v)
        plsc.subcore_barrier()

        def gather(step, buf):
            pltpu.async_copy(table_sh.at[idx_v.at[step]], rows_v.at[buf], gsem)

        def wait_gather(buf):
            pltpu.make_async_copy(
                table_sh.at[idx_v.at[0]], rows_v.at[buf], gsem
            ).wait()

        def wait_write(buf):
            pltpu.make_async_copy(
                rows_v.at[buf], out_hbm.at[pl.ds(base, CH)], wsems[buf]
            ).wait()

        for p in range(GA):
            gather(p, p)

        def group(g, carry):
            for b in range(NB):
                st = g * NB + b

                nb = (b + GA) % NB

                @pl.when(st + GA < n_steps)
                def _():

                    @pl.when(st + GA >= NB)
                    def _():
                        wait_write(nb)

                    gather(st + GA, nb)

                wait_gather(b)
                pltpu.async_copy(
                    rows_v.at[b], out_hbm.at[pl.ds(base + st * CH, CH)], wsems[b]
                )
            return carry

        lax.fori_loop(0, n_groups, group, 0)
        for b in range(NB):
            wait_write(b)

    return k, nc, nw, rows_per_w, n_steps


_sc_lookup, _NC, _NW, _RPW, _NSTEPS = _make_sc_lookup(TOTAL_ROWS, EMB)


def kernel(indices, table):
    B, T = indices.shape
    flat = indices.reshape(B * T).astype(jnp.int32)
    out = _sc_lookup(flat.reshape(_NW, _NSTEPS, CH), table)
    return out.reshape(B, T, EMB)
